# Initial kernel scaffold; baseline (speedup 1.0000x reference)
import jax
import jax.numpy as jnp
from jax.experimental import pallas as pl


def kernel(x, edge_index, edge_weight, batch, params):
    raise NotImplementedError("write your pallas kernel here")


# speedup vs baseline: 12.6346x; 12.6346x over previous
import functools

import jax
import jax.numpy as jnp
from jax import lax
from jax.experimental import pallas as pl
from jax.experimental.pallas import tpu as pltpu
from jax.experimental.pallas import tpu_sc as plsc

N = 10000
E = 320000
B = 64
H = 128
STEPS = 4
NC, NS = 2, 16
NW = NC * NS
C = 128
NCHUNK = E // C
KMAX = (NCHUNK + NW - 1) // NW

_HIGH = lax.Precision.HIGHEST


def _dot(a, b):
    return jnp.dot(a, b, precision=_HIGH, preferred_element_type=jnp.float32)


NZCH = N // C
NZTAIL = N - NZCH * C


@functools.lru_cache(maxsize=None)
def _make_spmm(W):
    mesh = plsc.VectorSubcoreMesh(core_axis_name="c", subcore_axis_name="s")

    @functools.partial(
        pl.kernel,
        mesh=mesh,
        compiler_params=pltpu.CompilerParams(use_tc_tiling_on_sc=False),
        out_type=jax.ShapeDtypeStruct((NC * N, W), jnp.float32),
        scratch_types=[
            pltpu.VMEM((C,), jnp.int32),
            pltpu.VMEM((C,), jnp.int32),
            pltpu.VMEM((C, W), jnp.float32),
            pltpu.VMEM_SHARED((N, W), jnp.float32),
            pltpu.SemaphoreType.DMA,
        ],
    )
    def spmm(h_hbm, sidx_hbm, didx_hbm, out_hbm, sbuf, dbuf, rows, spacc, sem):
        cid = lax.axis_index("c")
        sid = lax.axis_index("s")
        wid = sid * NC + cid
        z16 = jnp.zeros((16,), jnp.float32)

        def zrow(i, _):
            def zcol(j, _):
                rows[i, pl.ds(j * 16, 16)] = z16
                return 0

            return lax.fori_loop(0, W // 16, zcol, 0)

        lax.fori_loop(0, C, zrow, 0)

        def zchunk(k, _):
            t = k * NS + sid

            @pl.when(t < NZCH)
            def _():
                pltpu.sync_copy(rows, spacc.at[pl.ds(t * C, C)])

            return 0

        lax.fori_loop(0, (NZCH + NS - 1) // NS, zchunk, 0)

        @pl.when(sid == NS - 1)
        def _():
            pltpu.sync_copy(rows.at[pl.ds(0, NZTAIL)],
                            spacc.at[pl.ds(NZCH * C, NZTAIL)])

        plsc.subcore_barrier()

        def body(k, _):
            chunk = k * NW + wid

            @pl.when(chunk < NCHUNK)
            def _():
                pltpu.sync_copy(sidx_hbm.at[chunk], sbuf)
                pltpu.sync_copy(didx_hbm.at[chunk], dbuf)
                pltpu.async_copy(h_hbm.at[sbuf], rows, sem).wait()
                pltpu.sync_copy(rows, spacc.at[dbuf], add=True)

            return 0

        lax.fori_loop(0, KMAX, body, 0)
        plsc.subcore_barrier()

        def cchunk(k, _):
            t = k * NS + sid

            @pl.when(t < NZCH)
            def _():
                pltpu.sync_copy(spacc.at[pl.ds(t * C, C)],
                                out_hbm.at[pl.ds(cid * N + t * C, C)])

            return 0

        lax.fori_loop(0, (NZCH + NS - 1) // NS, cchunk, 0)

        @pl.when(sid == NS - 1)
        def _():
            pltpu.sync_copy(spacc.at[pl.ds(NZCH * C, NZTAIL)],
                            out_hbm.at[pl.ds(cid * N + NZCH * C, NZTAIL)])

    return spmm


def _pairsum_body(a_ref, b_ref, out_ref):
    out_ref[...] = a_ref[...] + b_ref[...]


def _pairsum(acc, W):
    blk = 1000
    g = N // blk
    return pl.pallas_call(
        _pairsum_body,
        grid=(g,),
        in_specs=[pl.BlockSpec((blk, W), lambda i: (i, 0)),
                  pl.BlockSpec((blk, W), lambda i: (i + g, 0))],
        out_specs=pl.BlockSpec((blk, W), lambda i: (i, 0)),
        out_shape=jax.ShapeDtypeStruct((N, W), jnp.float32),
    )(acc, acc)


def _bn(x, g, b, eps=1e-5):
    m = jnp.mean(x, axis=0, keepdims=True)
    xc = x - m
    v = jnp.mean(xc * xc, axis=0, keepdims=True)
    return xc * lax.rsqrt(v + eps) * g + b


def _elu(x):
    return jnp.where(x > 0, x, jnp.exp(x) - 1.0)


def _prep_body(x_ref, degp_ref, g0a, b0a, g0c, b0c, w1a, w1c, h1p_ref, dinv_ref):
    deg = degp_ref[pl.ds(0, N), pl.ds(0, 1)] + degp_ref[pl.ds(N, N), pl.ds(0, 1)] + 1.0
    dinv = lax.rsqrt(deg)
    x = x_ref[...]
    m = jnp.mean(x, axis=0, keepdims=True)
    xc = x - m
    v = jnp.mean(xc * xc, axis=0, keepdims=True)
    xn = xc * lax.rsqrt(v + 1e-5)
    ha = _dot(xn * g0a[...] + b0a[...], w1a[...])
    hc = _dot(xn * g0c[...] + b0c[...], w1c[...])
    h1p_ref[...] = jnp.concatenate([ha, hc], axis=1) * dinv
    dinv_ref[...] = dinv


def _tc_prep(x, degp, g0a, b0a, g0c, b0c, w1a, w1c):
    return pl.pallas_call(
        _prep_body,
        out_shape=[jax.ShapeDtypeStruct((N, 64), jnp.float32),
                   jax.ShapeDtypeStruct((N, 1), jnp.float32)],
    )(x, degp, g0a, b0a, g0c, b0c, w1a, w1c)


def _mid_body(acc_ref, hp_ref, dinv_ref, g_ref, b_ref, w_ref, out_ref):
    dinv = dinv_ref[...]
    pre = (acc_ref[...] + hp_ref[...]) * dinv
    z = _elu(_bn(pre, g_ref[...], b_ref[...]))
    out_ref[...] = _dot(z, w_ref[...]) * dinv


def _tc_mid(acc, hp, dinv, g, b, w):
    return pl.pallas_call(
        _mid_body,
        out_shape=jax.ShapeDtypeStruct((N, w.shape[1]), jnp.float32),
    )(acc, hp, dinv, g, b, w)


def _post3_body(acc_ref, hp_ref, dinv_ref, g_ref, b_ref, h_ref):
    pre = (acc_ref[...] + hp_ref[...]) * dinv_ref[...]
    h_ref[...] = _elu(_bn(pre, g_ref[...], b_ref[...]))


def _tc_post3(acc, hp, dinv, g, b):
    return pl.pallas_call(
        _post3_body,
        out_shape=jax.ShapeDtypeStruct((N, H), jnp.float32),
    )(acc, hp, dinv, g, b)


def _s2s_head(h, maskf, maskT, layers, d1w, dbng, dbnb, d2w, d2b):
    L = len(layers)
    hs = [jnp.zeros((B, H), jnp.float32) for _ in range(L)]
    cs = [jnp.zeros((B, H), jnp.float32) for _ in range(L)]
    q_star = jnp.zeros((B, 2 * H), jnp.float32)
    for _ in range(STEPS):
        inp = q_star
        nh, ncs = [], []
        for l, (wih, whh, bias) in enumerate(layers):
            gates = _dot(inp, wih) + _dot(hs[l], whh) + bias
            gi = jax.nn.sigmoid(gates[:, 0 * H:1 * H])
            gf = jax.nn.sigmoid(gates[:, 1 * H:2 * H])
            gg = jnp.tanh(gates[:, 2 * H:3 * H])
            go = jax.nn.sigmoid(gates[:, 3 * H:4 * H])
            c = gf * cs[l] + gi * gg
            hl = go * jnp.tanh(c)
            nh.append(hl)
            ncs.append(c)
            inp = hl
        hs, cs = nh, ncs
        q = inp
        qb = _dot(maskT, q)
        e = jnp.sum(h * qb, axis=1, keepdims=True)
        em = jnp.where(maskf > 0, e.reshape(1, N), -jnp.inf)
        m = jnp.max(em, axis=1, keepdims=True)
        m = jnp.where(m > -jnp.inf, m, 0.0)
        ex = jnp.exp(e - _dot(maskT, m))
        s = _dot(maskf, ex)
        a = ex / (_dot(maskT, s) + 1e-16)
        r = _dot(maskf, a * h)
        q_star = jnp.concatenate([q, r], axis=1)
    z = _elu(_bn(_dot(q_star, d1w), dbng, dbnb))
    return _dot(z, d2w) + d2b


def _make_s2s_body(L):
    def body(h_ref, brow_ref, bcol_ref, *refs):
        lstm = [(refs[3 * l][...], refs[3 * l + 1][...], refs[3 * l + 2][...])
                for l in range(L)]
        d1w, dbng, dbnb, d2w, d2b = (r[...] for r in refs[3 * L:3 * L + 5])
        out_ref = refs[3 * L + 5]
        gid_row = lax.broadcasted_iota(jnp.int32, (B, N), 0)
        gid_col = lax.broadcasted_iota(jnp.int32, (N, B), 1)
        maskf = (gid_row == brow_ref[...]).astype(jnp.float32)
        maskT = (gid_col == bcol_ref[...]).astype(jnp.float32)
        out_ref[...] = _s2s_head(h_ref[...], maskf, maskT, lstm,
                                 d1w, dbng, dbnb, d2w, d2b)

    return body


def _tc_s2s(h, brow, bcol, lstm_args, head_args):
    L = len(lstm_args) // 3
    return pl.pallas_call(
        _make_s2s_body(L),
        out_shape=jax.ShapeDtypeStruct((B, 32), jnp.float32),
    )(h, brow, bcol, *lstm_args, *head_args)


def kernel(x, edge_index, edge_weight, batch, params):
    pa, pc = params["a"], params["c"]
    src = edge_index[0].reshape(NCHUNK, C)
    dst = edge_index[1].reshape(NCHUNK, C)
    brow = batch.reshape(1, N)
    bcol = batch.reshape(N, 1)

    r1 = lambda a: a.reshape(1, -1)

    degp = _make_spmm(16)(jnp.ones((N, 16), jnp.float32), src, dst)

    h1p, dinv = _tc_prep(x, degp,
                         r1(pa["bn0_g"]), r1(pa["bn0_b"]),
                         r1(pc["bn0_g"]), r1(pc["bn0_b"]),
                         pa["W1"], pc["W1"])

    acc1 = _pairsum(_make_spmm(64)(h1p, src, dst), 64)
    h2pa = _tc_mid(acc1[:, :32], h1p[:, :32], dinv,
                   r1(pa["bn1_g"]), r1(pa["bn1_b"]), pa["W2"])
    h2pc = _tc_mid(acc1[:, 32:], h1p[:, 32:], dinv,
                   r1(pc["bn1_g"]), r1(pc["bn1_b"]), pc["W2"])
    h2p = jnp.concatenate([h2pa, h2pc], axis=1)

    acc2 = _pairsum(_make_spmm(128)(h2p, src, dst), 128)
    h3pa = _tc_mid(acc2[:, :64], h2pa, dinv,
                   r1(pa["bn2_g"]), r1(pa["bn2_b"]), pa["W3"])
    h3pc = _tc_mid(acc2[:, 64:], h2pc, dinv,
                   r1(pc["bn2_g"]), r1(pc["bn2_b"]), pc["W3"])

    acc3a = _pairsum(_make_spmm(128)(h3pa, src, dst), 128)
    acc3c = _pairsum(_make_spmm(128)(h3pc, src, dst), 128)

    ha = _tc_post3(acc3a, h3pa, dinv, r1(pa["bn3_g"]), r1(pa["bn3_b"]))
    hc = _tc_post3(acc3c, h3pc, dinv, r1(pc["bn3_g"]), r1(pc["bn3_b"]))

    la = pa["lstm"][0]
    lc0, lc1 = pc["lstm"][0], pc["lstm"][1]
    outa = _tc_s2s(ha, brow, bcol,
                   [la["Wih"].T, la["Whh"].T, r1(la["bih"] + la["bhh"])],
                   [pa["dnn1_W"], r1(pa["dnn_bn_g"]), r1(pa["dnn_bn_b"]),
                    pa["dnn2_W"], r1(pa["dnn2_b"])])
    outc = _tc_s2s(hc, brow, bcol,
                   [lc0["Wih"].T, lc0["Whh"].T, r1(lc0["bih"] + lc0["bhh"]),
                    lc1["Wih"].T, lc1["Whh"].T, r1(lc1["bih"] + lc1["bhh"])],
                   [pc["dnn1_W"], r1(pc["dnn_bn_g"]), r1(pc["dnn_bn_b"]),
                    pc["dnn2_W"], r1(pc["dnn2_b"])])
    return (outa, outc)


# speedup vs baseline: 20.4323x; 1.6172x over previous
import functools

import jax
import jax.numpy as jnp
from jax import lax
from jax.experimental import pallas as pl
from jax.experimental.pallas import tpu as pltpu
from jax.experimental.pallas import tpu_sc as plsc

N = 10000
E = 320000
B = 64
H = 128
STEPS = 4
NC, NS = 2, 16
NW = NC * NS
C = 128
NCHUNK = E // C
KMAX = (NCHUNK + NW - 1) // NW

_HIGH = lax.Precision.HIGHEST


def _dot(a, b):
    return jnp.dot(a, b, precision=_HIGH, preferred_element_type=jnp.float32)


NZCH = N // C
NZTAIL = N - NZCH * C


@functools.lru_cache(maxsize=None)
def _make_spmm(W):
    mesh = plsc.VectorSubcoreMesh(core_axis_name="c", subcore_axis_name="s")

    @functools.partial(
        pl.kernel,
        mesh=mesh,
        compiler_params=pltpu.CompilerParams(use_tc_tiling_on_sc=False),
        out_type=jax.ShapeDtypeStruct((NC * N, W), jnp.float32),
        scratch_types=[
            pltpu.VMEM((2, C), jnp.int32),
            pltpu.VMEM((2, C), jnp.int32),
            pltpu.VMEM((C, W), jnp.float32),
            pltpu.VMEM((C, W), jnp.float32),
            pltpu.VMEM_SHARED((N, W), jnp.float32),
            pltpu.SemaphoreType.DMA,
            pltpu.SemaphoreType.DMA,
        ],
    )
    def spmm(h_hbm, eidx_hbm, out_hbm, ibufa, ibufb, rowsa, rowsb, spacc,
             sema, semb):
        cid = lax.axis_index("c")
        sid = lax.axis_index("s")
        wid = sid * NC + cid
        z16 = jnp.zeros((16,), jnp.float32)

        def zrow(i, _):
            def zcol(j, _):
                rowsa[i, pl.ds(j * 16, 16)] = z16
                return 0

            return lax.fori_loop(0, W // 16, zcol, 0)

        lax.fori_loop(0, C, zrow, 0)

        def zchunk(k, _):
            t = k * NS + sid

            @pl.when(t < NZCH)
            def _():
                pltpu.sync_copy(rowsa, spacc.at[pl.ds(t * C, C)])

            return 0

        lax.fori_loop(0, (NZCH + NS - 1) // NS, zchunk, 0)

        @pl.when(sid == NS - 1)
        def _():
            pltpu.sync_copy(rowsa.at[pl.ds(0, NZTAIL)],
                            spacc.at[pl.ds(NZCH * C, NZTAIL)])

        plsc.subcore_barrier()

        def idx_copy(k, ibuf):
            pltpu.sync_copy(eidx_hbm.at[k * NW + wid], ibuf)

        def gather_start(ibuf, rows, sem):
            pltpu.make_async_copy(h_hbm.at[ibuf.at[0]], rows, sem).start()

        def gather_wait(ibuf, rows, sem):
            pltpu.make_async_copy(h_hbm.at[ibuf.at[0]], rows, sem).wait()

        def scatter(ibuf, rows):
            pltpu.sync_copy(rows, spacc.at[ibuf.at[1]], add=True)

        idx_copy(0, ibufa)
        gather_start(ibufa, rowsa, sema)

        def body(t, _):
            k0 = 2 * t
            c0 = k0 * NW + wid
            c1 = c0 + NW
            c2 = c1 + NW

            @pl.when(c1 < NCHUNK)
            def _():
                idx_copy(k0 + 1, ibufb)
                gather_start(ibufb, rowsb, semb)

            @pl.when(c0 < NCHUNK)
            def _():
                gather_wait(ibufa, rowsa, sema)
                scatter(ibufa, rowsa)

            @pl.when(c2 < NCHUNK)
            def _():
                idx_copy(k0 + 2, ibufa)
                gather_start(ibufa, rowsa, sema)

            @pl.when(c1 < NCHUNK)
            def _():
                gather_wait(ibufb, rowsb, semb)
                scatter(ibufb, rowsb)

            return 0

        lax.fori_loop(0, (KMAX + 1) // 2, body, 0)
        plsc.subcore_barrier()

        def cchunk(k, _):
            t = k * NS + sid

            @pl.when(t < NZCH)
            def _():
                pltpu.sync_copy(spacc.at[pl.ds(t * C, C)],
                                out_hbm.at[pl.ds(cid * N + t * C, C)])

            return 0

        lax.fori_loop(0, (NZCH + NS - 1) // NS, cchunk, 0)

        @pl.when(sid == NS - 1)
        def _():
            pltpu.sync_copy(spacc.at[pl.ds(NZCH * C, NZTAIL)],
                            out_hbm.at[pl.ds(cid * N + NZCH * C, NZTAIL)])

    return spmm


def _pairsum_body(a_ref, b_ref, out_ref):
    out_ref[...] = a_ref[...] + b_ref[...]


def _pairsum(acc, W):
    blk = 1000
    g = N // blk
    return pl.pallas_call(
        _pairsum_body,
        grid=(g,),
        in_specs=[pl.BlockSpec((blk, W), lambda i: (i, 0)),
                  pl.BlockSpec((blk, W), lambda i: (i + g, 0))],
        out_specs=pl.BlockSpec((blk, W), lambda i: (i, 0)),
        out_shape=jax.ShapeDtypeStruct((N, W), jnp.float32),
    )(acc, acc)


def _bn(x, g, b, eps=1e-5):
    m = jnp.mean(x, axis=0, keepdims=True)
    xc = x - m
    v = jnp.mean(xc * xc, axis=0, keepdims=True)
    return xc * lax.rsqrt(v + eps) * g + b


def _elu(x):
    return jnp.where(x > 0, x, jnp.exp(x) - 1.0)


def _prep_body(x_ref, degp_ref, g0a, b0a, g0c, b0c, w1a, w1c, h1p_ref, dinv_ref):
    deg = degp_ref[pl.ds(0, N), pl.ds(0, 1)] + degp_ref[pl.ds(N, N), pl.ds(0, 1)] + 1.0
    dinv = lax.rsqrt(deg)
    x = x_ref[...]
    m = jnp.mean(x, axis=0, keepdims=True)
    xc = x - m
    v = jnp.mean(xc * xc, axis=0, keepdims=True)
    xn = xc * lax.rsqrt(v + 1e-5)
    ha = _dot(xn * g0a[...] + b0a[...], w1a[...])
    hc = _dot(xn * g0c[...] + b0c[...], w1c[...])
    h1p_ref[...] = jnp.concatenate([ha, hc], axis=1) * dinv
    dinv_ref[...] = dinv


def _tc_prep(x, degp, g0a, b0a, g0c, b0c, w1a, w1c):
    return pl.pallas_call(
        _prep_body,
        out_shape=[jax.ShapeDtypeStruct((N, 64), jnp.float32),
                   jax.ShapeDtypeStruct((N, 1), jnp.float32)],
    )(x, degp, g0a, b0a, g0c, b0c, w1a, w1c)


def _mid_body(acc_ref, hp_ref, dinv_ref, g_ref, b_ref, w_ref, out_ref):
    dinv = dinv_ref[...]
    pre = (acc_ref[...] + hp_ref[...]) * dinv
    z = _elu(_bn(pre, g_ref[...], b_ref[...]))
    out_ref[...] = _dot(z, w_ref[...]) * dinv


def _tc_mid(acc, hp, dinv, g, b, w):
    return pl.pallas_call(
        _mid_body,
        out_shape=jax.ShapeDtypeStruct((N, w.shape[1]), jnp.float32),
    )(acc, hp, dinv, g, b, w)


def _post3_body(acc_ref, hp_ref, dinv_ref, g_ref, b_ref, h_ref):
    pre = (acc_ref[...] + hp_ref[...]) * dinv_ref[...]
    h_ref[...] = _elu(_bn(pre, g_ref[...], b_ref[...]))


def _tc_post3(acc, hp, dinv, g, b):
    return pl.pallas_call(
        _post3_body,
        out_shape=jax.ShapeDtypeStruct((N, H), jnp.float32),
    )(acc, hp, dinv, g, b)


def _s2s_head(h, maskf, maskT, layers, d1w, dbng, dbnb, d2w, d2b):
    L = len(layers)
    hs = [jnp.zeros((B, H), jnp.float32) for _ in range(L)]
    cs = [jnp.zeros((B, H), jnp.float32) for _ in range(L)]
    q_star = jnp.zeros((B, 2 * H), jnp.float32)
    for _ in range(STEPS):
        inp = q_star
        nh, ncs = [], []
        for l, (wih, whh, bias) in enumerate(layers):
            gates = _dot(inp, wih) + _dot(hs[l], whh) + bias
            gi = jax.nn.sigmoid(gates[:, 0 * H:1 * H])
            gf = jax.nn.sigmoid(gates[:, 1 * H:2 * H])
            gg = jnp.tanh(gates[:, 2 * H:3 * H])
            go = jax.nn.sigmoid(gates[:, 3 * H:4 * H])
            c = gf * cs[l] + gi * gg
            hl = go * jnp.tanh(c)
            nh.append(hl)
            ncs.append(c)
            inp = hl
        hs, cs = nh, ncs
        q = inp
        qb = _dot(maskT, q)
        e = jnp.sum(h * qb, axis=1, keepdims=True)
        em = jnp.where(maskf > 0, e.reshape(1, N), -jnp.inf)
        m = jnp.max(em, axis=1, keepdims=True)
        m = jnp.where(m > -jnp.inf, m, 0.0)
        ex = jnp.exp(e - _dot(maskT, m))
        s = _dot(maskf, ex)
        a = ex / (_dot(maskT, s) + 1e-16)
        r = _dot(maskf, a * h)
        q_star = jnp.concatenate([q, r], axis=1)
    z = _elu(_bn(_dot(q_star, d1w), dbng, dbnb))
    return _dot(z, d2w) + d2b


def _make_s2s_body(L):
    def body(h_ref, brow_ref, bcol_ref, *refs):
        lstm = [(refs[3 * l][...], refs[3 * l + 1][...], refs[3 * l + 2][...])
                for l in range(L)]
        d1w, dbng, dbnb, d2w, d2b = (r[...] for r in refs[3 * L:3 * L + 5])
        out_ref = refs[3 * L + 5]
        gid_row = lax.broadcasted_iota(jnp.int32, (B, N), 0)
        gid_col = lax.broadcasted_iota(jnp.int32, (N, B), 1)
        maskf = (gid_row == brow_ref[...]).astype(jnp.float32)
        maskT = (gid_col == bcol_ref[...]).astype(jnp.float32)
        out_ref[...] = _s2s_head(h_ref[...], maskf, maskT, lstm,
                                 d1w, dbng, dbnb, d2w, d2b)

    return body


def _tc_s2s(h, brow, bcol, lstm_args, head_args):
    L = len(lstm_args) // 3
    return pl.pallas_call(
        _make_s2s_body(L),
        out_shape=jax.ShapeDtypeStruct((B, 32), jnp.float32),
    )(h, brow, bcol, *lstm_args, *head_args)


def kernel(x, edge_index, edge_weight, batch, params):
    pa, pc = params["a"], params["c"]
    eidx = edge_index.reshape(2, NCHUNK, C).transpose(1, 0, 2)
    brow = batch.reshape(1, N)
    bcol = batch.reshape(N, 1)

    r1 = lambda a: a.reshape(1, -1)

    degp = _make_spmm(16)(jnp.ones((N, 16), jnp.float32), eidx)

    h1p, dinv = _tc_prep(x, degp,
                         r1(pa["bn0_g"]), r1(pa["bn0_b"]),
                         r1(pc["bn0_g"]), r1(pc["bn0_b"]),
                         pa["W1"], pc["W1"])

    acc1 = _pairsum(_make_spmm(64)(h1p, eidx), 64)
    h2pa = _tc_mid(acc1[:, :32], h1p[:, :32], dinv,
                   r1(pa["bn1_g"]), r1(pa["bn1_b"]), pa["W2"])
    h2pc = _tc_mid(acc1[:, 32:], h1p[:, 32:], dinv,
                   r1(pc["bn1_g"]), r1(pc["bn1_b"]), pc["W2"])
    h2p = jnp.concatenate([h2pa, h2pc], axis=1)

    acc2 = _pairsum(_make_spmm(128)(h2p, eidx), 128)
    h3pa = _tc_mid(acc2[:, :64], h2pa, dinv,
                   r1(pa["bn2_g"]), r1(pa["bn2_b"]), pa["W3"])
    h3pc = _tc_mid(acc2[:, 64:], h2pc, dinv,
                   r1(pc["bn2_g"]), r1(pc["bn2_b"]), pc["W3"])

    acc3a = _pairsum(_make_spmm(128)(h3pa, eidx), 128)
    acc3c = _pairsum(_make_spmm(128)(h3pc, eidx), 128)

    ha = _tc_post3(acc3a, h3pa, dinv, r1(pa["bn3_g"]), r1(pa["bn3_b"]))
    hc = _tc_post3(acc3c, h3pc, dinv, r1(pc["bn3_g"]), r1(pc["bn3_b"]))

    la = pa["lstm"][0]
    lc0, lc1 = pc["lstm"][0], pc["lstm"][1]
    outa = _tc_s2s(ha, brow, bcol,
                   [la["Wih"].T, la["Whh"].T, r1(la["bih"] + la["bhh"])],
                   [pa["dnn1_W"], r1(pa["dnn_bn_g"]), r1(pa["dnn_bn_b"]),
                    pa["dnn2_W"], r1(pa["dnn2_b"])])
    outc = _tc_s2s(hc, brow, bcol,
                   [lc0["Wih"].T, lc0["Whh"].T, r1(lc0["bih"] + lc0["bhh"]),
                    lc1["Wih"].T, lc1["Whh"].T, r1(lc1["bih"] + lc1["bhh"])],
                   [pc["dnn1_W"], r1(pc["dnn_bn_g"]), r1(pc["dnn_bn_b"]),
                    pc["dnn2_W"], r1(pc["dnn2_b"])])
    return (outa, outc)


# speedup vs baseline: 23.3739x; 1.1440x over previous
import functools

import jax
import jax.numpy as jnp
from jax import lax
from jax.experimental import pallas as pl
from jax.experimental.pallas import tpu as pltpu
from jax.experimental.pallas import tpu_sc as plsc

N = 10000
E = 320000
B = 64
H = 128
STEPS = 4
NC, NS = 2, 16
NW = NC * NS
C = 128

_HIGH = lax.Precision.HIGHEST


def _dot(a, b):
    return jnp.dot(a, b, precision=_HIGH, preferred_element_type=jnp.float32)


def _spmm_chunk(W):
    return 64 if W >= 128 else 128


@functools.lru_cache(maxsize=None)
def _make_spmm(W):
    mesh = plsc.VectorSubcoreMesh(core_axis_name="c", subcore_axis_name="s")
    c = _spmm_chunk(W)
    nchunk = E // c
    kmax = (nchunk + NW - 1) // NW
    nzch = N // c
    nztail = N - nzch * c

    @functools.partial(
        pl.kernel,
        mesh=mesh,
        compiler_params=pltpu.CompilerParams(use_tc_tiling_on_sc=False),
        out_type=jax.ShapeDtypeStruct((NC * N, W), jnp.float32),
        scratch_types=(
            [pltpu.VMEM((2, c), jnp.int32)] * 4
            + [pltpu.VMEM((c, W), jnp.float32)] * 4
            + [pltpu.VMEM_SHARED((N, W), jnp.float32)]
            + [pltpu.SemaphoreType.DMA] * 8
        ),
    )
    def spmm(h_hbm, eidx_hbm, out_hbm, *sc):
        ibuf = sc[0:4]
        rows = sc[4:8]
        spacc = sc[8]
        gsem = sc[9:13]
        ssem = sc[13:17]
        rowsa = rows[0]
        cid = lax.axis_index("c")
        sid = lax.axis_index("s")
        wid = sid * NC + cid
        z16 = jnp.zeros((16,), jnp.float32)

        def zrow(i, _):
            def zcol(j, _):
                rowsa[i, pl.ds(j * 16, 16)] = z16
                return 0

            return lax.fori_loop(0, W // 16, zcol, 0)

        lax.fori_loop(0, c, zrow, 0)

        def zchunk(k, _):
            t = k * NS + sid

            @pl.when(t < nzch)
            def _():
                pltpu.sync_copy(rowsa, spacc.at[pl.ds(t * c, c)])

            return 0

        lax.fori_loop(0, (nzch + NS - 1) // NS, zchunk, 0)

        @pl.when(sid == NS - 1)
        def _():
            pltpu.sync_copy(rowsa.at[pl.ds(0, nztail)],
                            spacc.at[pl.ds(nzch * c, nztail)])

        plsc.subcore_barrier()

        def idx_copy(k, j):
            pltpu.sync_copy(eidx_hbm.at[k * NW + wid], ibuf[j])

        def gather_start(j):
            pltpu.make_async_copy(h_hbm.at[ibuf[j].at[0]], rows[j],
                                  gsem[j]).start()

        def gather_wait(j):
            pltpu.make_async_copy(h_hbm.at[ibuf[j].at[0]], rows[j],
                                  gsem[j]).wait()

        def scatter_start(j):
            pltpu.async_copy(rows[j], spacc.at[ibuf[j].at[1]], ssem[j],
                             add=True)

        def scatter_wait(j):
            pltpu.make_async_copy(rows[j], spacc.at[ibuf[j].at[1]],
                                  ssem[j]).wait()

        for k in (0, 1):
            idx_copy(k, k)
            gather_start(k)

        def body(t, _):
            for j in range(4):
                k = 4 * t + j
                ck = k * NW + wid
                cm = ck + 2 * NW
                jm = (j + 2) % 4

                @pl.when(cm < nchunk)
                def _():
                    @pl.when(k >= 2)
                    def _():
                        scatter_wait(jm)

                    idx_copy(k + 2, jm)
                    gather_start(jm)

                @pl.when(ck < nchunk)
                def _():
                    gather_wait(j)
                    scatter_start(j)

            return 0

        lax.fori_loop(0, (kmax + 3) // 4, body, 0)
        for j in range(4):
            scatter_wait(j)
        plsc.subcore_barrier()

        def cchunk(k, _):
            t = k * NS + sid

            @pl.when(t < nzch)
            def _():
                pltpu.sync_copy(spacc.at[pl.ds(t * c, c)],
                                out_hbm.at[pl.ds(cid * N + t * c, c)])

            return 0

        lax.fori_loop(0, (nzch + NS - 1) // NS, cchunk, 0)

        @pl.when(sid == NS - 1)
        def _():
            pltpu.sync_copy(spacc.at[pl.ds(nzch * c, nztail)],
                            out_hbm.at[pl.ds(cid * N + nzch * c, nztail)])

    return spmm


def _bn(x, g, b, eps=1e-5):
    m = jnp.mean(x, axis=0, keepdims=True)
    xc = x - m
    v = jnp.mean(xc * xc, axis=0, keepdims=True)
    return xc * lax.rsqrt(v + eps) * g + b


def _elu(x):
    return jnp.where(x > 0, x, jnp.exp(x) - 1.0)


def _prep_body(x_ref, degp_ref, g0a, b0a, g0c, b0c, w1a, w1c, h1p_ref, dinv_ref):
    deg = degp_ref[pl.ds(0, N), pl.ds(0, 1)] + degp_ref[pl.ds(N, N), pl.ds(0, 1)] + 1.0
    dinv = lax.rsqrt(deg)
    x = x_ref[...]
    m = jnp.mean(x, axis=0, keepdims=True)
    xc = x - m
    v = jnp.mean(xc * xc, axis=0, keepdims=True)
    xn = xc * lax.rsqrt(v + 1e-5)
    ha = _dot(xn * g0a[...] + b0a[...], w1a[...])
    hc = _dot(xn * g0c[...] + b0c[...], w1c[...])
    h1p_ref[...] = jnp.concatenate([ha, hc], axis=1) * dinv
    dinv_ref[...] = dinv


def _tc_prep(x, degp, g0a, b0a, g0c, b0c, w1a, w1c):
    return pl.pallas_call(
        _prep_body,
        out_shape=[jax.ShapeDtypeStruct((N, 64), jnp.float32),
                   jax.ShapeDtypeStruct((N, 1), jnp.float32)],
    )(x, degp, g0a, b0a, g0c, b0c, w1a, w1c)


def _make_mid_body(off, wb):
    def body(acc_ref, hp_ref, dinv_ref, g_ref, b_ref, w_ref, out_ref):
        dinv = dinv_ref[...]
        accb = (acc_ref[pl.ds(0, N), pl.ds(off, wb)]
                + acc_ref[pl.ds(N, N), pl.ds(off, wb)])
        pre = (accb + hp_ref[...]) * dinv
        z = _elu(_bn(pre, g_ref[...], b_ref[...]))
        out_ref[...] = _dot(z, w_ref[...]) * dinv

    return body


def _tc_mid(acc, off, hp, dinv, g, b, w):
    return pl.pallas_call(
        _make_mid_body(off, hp.shape[1]),
        out_shape=jax.ShapeDtypeStruct((N, w.shape[1]), jnp.float32),
    )(acc, hp, dinv, g, b, w)


def _post3_body(acc_ref, hp_ref, dinv_ref, g_ref, b_ref, h_ref):
    accb = acc_ref[pl.ds(0, N), :] + acc_ref[pl.ds(N, N), :]
    pre = (accb + hp_ref[...]) * dinv_ref[...]
    h_ref[...] = _elu(_bn(pre, g_ref[...], b_ref[...]))


def _tc_post3(acc, hp, dinv, g, b):
    return pl.pallas_call(
        _post3_body,
        out_shape=jax.ShapeDtypeStruct((N, H), jnp.float32),
    )(acc, hp, dinv, g, b)


def _s2s_head(h, maskf, maskT, layers, d1w, dbng, dbnb, d2w, d2b):
    L = len(layers)
    hs = [jnp.zeros((B, H), jnp.float32) for _ in range(L)]
    cs = [jnp.zeros((B, H), jnp.float32) for _ in range(L)]
    q_star = jnp.zeros((B, 2 * H), jnp.float32)
    for _ in range(STEPS):
        inp = q_star
        nh, ncs = [], []
        for l, (wih, whh, bias) in enumerate(layers):
            gates = _dot(inp, wih) + _dot(hs[l], whh) + bias
            gi = jax.nn.sigmoid(gates[:, 0 * H:1 * H])
            gf = jax.nn.sigmoid(gates[:, 1 * H:2 * H])
            gg = jnp.tanh(gates[:, 2 * H:3 * H])
            go = jax.nn.sigmoid(gates[:, 3 * H:4 * H])
            c = gf * cs[l] + gi * gg
            hl = go * jnp.tanh(c)
            nh.append(hl)
            ncs.append(c)
            inp = hl
        hs, cs = nh, ncs
        q = inp
        qb = _dot(maskT, q)
        e = jnp.sum(h * qb, axis=1, keepdims=True)
        em = jnp.where(maskf > 0, e.reshape(1, N), -jnp.inf)
        m = jnp.max(em, axis=1, keepdims=True)
        m = jnp.where(m > -jnp.inf, m, 0.0)
        ex = jnp.exp(e - _dot(maskT, m))
        s = _dot(maskf, ex)
        a = ex / (_dot(maskT, s) + 1e-16)
        r = _dot(maskf, a * h)
        q_star = jnp.concatenate([q, r], axis=1)
    z = _elu(_bn(_dot(q_star, d1w), dbng, dbnb))
    return _dot(z, d2w) + d2b


def _make_s2s_body(L):
    def body(h_ref, brow_ref, bcol_ref, *refs):
        lstm = [(refs[3 * l][...], refs[3 * l + 1][...], refs[3 * l + 2][...])
                for l in range(L)]
        d1w, dbng, dbnb, d2w, d2b = (r[...] for r in refs[3 * L:3 * L + 5])
        out_ref = refs[3 * L + 5]
        gid_row = lax.broadcasted_iota(jnp.int32, (B, N), 0)
        gid_col = lax.broadcasted_iota(jnp.int32, (N, B), 1)
        maskf = (gid_row == brow_ref[...]).astype(jnp.float32)
        maskT = (gid_col == bcol_ref[...]).astype(jnp.float32)
        out_ref[...] = _s2s_head(h_ref[...], maskf, maskT, lstm,
                                 d1w, dbng, dbnb, d2w, d2b)

    return body


def _tc_s2s(h, brow, bcol, lstm_args, head_args):
    L = len(lstm_args) // 3
    return pl.pallas_call(
        _make_s2s_body(L),
        out_shape=jax.ShapeDtypeStruct((B, 32), jnp.float32),
    )(h, brow, bcol, *lstm_args, *head_args)


def kernel(x, edge_index, edge_weight, batch, params):
    pa, pc = params["a"], params["c"]
    eidx128 = edge_index.reshape(2, E // 128, 128).transpose(1, 0, 2)
    eidx64 = edge_index.reshape(2, E // 64, 64).transpose(1, 0, 2)
    brow = batch.reshape(1, N)
    bcol = batch.reshape(N, 1)

    r1 = lambda a: a.reshape(1, -1)

    degp = _make_spmm(16)(jnp.ones((N, 16), jnp.float32), eidx128)

    h1p, dinv = _tc_prep(x, degp,
                         r1(pa["bn0_g"]), r1(pa["bn0_b"]),
                         r1(pc["bn0_g"]), r1(pc["bn0_b"]),
                         pa["W1"], pc["W1"])

    acc1 = _make_spmm(64)(h1p, eidx128)
    h2pa = _tc_mid(acc1, 0, h1p[:, :32], dinv,
                   r1(pa["bn1_g"]), r1(pa["bn1_b"]), pa["W2"])
    h2pc = _tc_mid(acc1, 32, h1p[:, 32:], dinv,
                   r1(pc["bn1_g"]), r1(pc["bn1_b"]), pc["W2"])
    h2p = jnp.concatenate([h2pa, h2pc], axis=1)

    acc2 = _make_spmm(128)(h2p, eidx64)
    h3pa = _tc_mid(acc2, 0, h2pa, dinv,
                   r1(pa["bn2_g"]), r1(pa["bn2_b"]), pa["W3"])
    h3pc = _tc_mid(acc2, 64, h2pc, dinv,
                   r1(pc["bn2_g"]), r1(pc["bn2_b"]), pc["W3"])

    acc3a = _make_spmm(128)(h3pa, eidx64)
    acc3c = _make_spmm(128)(h3pc, eidx64)

    ha = _tc_post3(acc3a, h3pa, dinv, r1(pa["bn3_g"]), r1(pa["bn3_b"]))
    hc = _tc_post3(acc3c, h3pc, dinv, r1(pc["bn3_g"]), r1(pc["bn3_b"]))

    la = pa["lstm"][0]
    lc0, lc1 = pc["lstm"][0], pc["lstm"][1]
    outa = _tc_s2s(ha, brow, bcol,
                   [la["Wih"].T, la["Whh"].T, r1(la["bih"] + la["bhh"])],
                   [pa["dnn1_W"], r1(pa["dnn_bn_g"]), r1(pa["dnn_bn_b"]),
                    pa["dnn2_W"], r1(pa["dnn2_b"])])
    outc = _tc_s2s(hc, brow, bcol,
                   [lc0["Wih"].T, lc0["Whh"].T, r1(lc0["bih"] + lc0["bhh"]),
                    lc1["Wih"].T, lc1["Whh"].T, r1(lc1["bih"] + lc1["bhh"])],
                   [pc["dnn1_W"], r1(pc["dnn_bn_g"]), r1(pc["dnn_bn_b"]),
                    pc["dnn2_W"], r1(pc["dnn2_b"])])
    return (outa, outc)
